# baseline (speedup 1.0000x reference)
import functools

import jax
import jax.numpy as jnp
from jax import lax
from jax.experimental import pallas as pl
from jax.experimental.pallas import tpu as pltpu

NH = 12
HD = 64

_I = False


def _rms(x, scale):
    return x * lax.rsqrt(jnp.mean(x * x, axis=-1, keepdims=True) + 1e-6) * scale


def _emb_gather(table, idx_flat, rows_per_step=8):
    T = idx_flat.shape[0]
    V, D = table.shape
    R = rows_per_step
    table3 = table.reshape(V, 1, D)

    def body(idx_ref, *refs):
        o_ref = refs[-1]
        for j in range(R):
            o_ref[j, :] = refs[j][0, 0, :]

    grid_spec = pltpu.PrefetchScalarGridSpec(
        num_scalar_prefetch=1,
        grid=(T // R,),
        in_specs=[
            pl.BlockSpec((1, 1, D), functools.partial(
                lambda j, i, idx_ref: (idx_ref[i * R + j], 0, 0), j))
            for j in range(R)
        ],
        out_specs=pl.BlockSpec((R, D), lambda i, idx_ref: (i, 0)),
    )
    return pl.pallas_call(
        body, grid_spec=grid_spec,
        out_shape=jax.ShapeDtypeStruct((T, D), table.dtype),
        interpret=_I,
    )(idx_flat, *([table3] * R))


def _qkv_body(x_ref, sc_ref, c_ref, s_ref, wqk_ref, wqks_ref, wv_ref,
              bqk_ref, bv_ref, q_ref, k_ref, v_ref):
    D = x_ref.shape[1]
    xn = _rms(x_ref[...], sc_ref[...])
    a = xn * c_ref[...]
    b = xn * s_ref[...]
    dn = (((1,), (1,)), ((), ()))
    qk = (lax.dot_general(a, wqk_ref[...], dn, preferred_element_type=jnp.float32)
          + lax.dot_general(b, wqks_ref[...], dn, preferred_element_type=jnp.float32)
          + bqk_ref[...])
    q_ref[...] = qk[:, :D]
    k_ref[...] = qk[:, D:]
    v_ref[...] = (lax.dot_general(xn, wv_ref[...], dn,
                                  preferred_element_type=jnp.float32) + bv_ref[...])


def _qkv(x, scale, C, S2, wqk, wqks, wv, bqk, bv, blk=512):
    T, D = x.shape
    out = jax.ShapeDtypeStruct((T, D), jnp.float32)
    return pl.pallas_call(
        _qkv_body,
        grid=(T // blk,),
        in_specs=[
            pl.BlockSpec((blk, D), lambda i: (i, 0)),
            pl.BlockSpec((D,), lambda i: (0,)),
            pl.BlockSpec((blk, D), lambda i: (i, 0)),
            pl.BlockSpec((blk, D), lambda i: (i, 0)),
            pl.BlockSpec((2 * D, D), lambda i: (0, 0)),
            pl.BlockSpec((2 * D, D), lambda i: (0, 0)),
            pl.BlockSpec((D, D), lambda i: (0, 0)),
            pl.BlockSpec((2 * D,), lambda i: (0,)),
            pl.BlockSpec((D,), lambda i: (0,)),
        ],
        out_specs=[pl.BlockSpec((blk, D), lambda i: (i, 0))] * 3,
        out_shape=[out, out, out],
        interpret=_I,
    )(x, scale, C, S2, wqk, wqks, wv, bqk, bv)


def _attn_body(q_ref, k_ref, v_ref, o_ref, *, half, bq, win):
    qb = pl.program_id(1)
    q = q_ref[0]
    ntile = half // bq
    start = jnp.maximum(qb - ntile, 0) * bq
    kw = k_ref[0, pl.ds(start, win), :]
    vw = v_ref[0, pl.ds(start, win), :]
    dn = (((1,), (1,)), ((), ()))
    s = lax.dot_general(q, kw, dn, preferred_element_type=jnp.float32) * (HD ** -0.5)
    rows = qb * bq + lax.broadcasted_iota(jnp.int32, (bq, win), 0)
    cols = start + lax.broadcasted_iota(jnp.int32, (bq, win), 1)
    bad = (cols > rows) | (cols <= rows - half)
    s = jnp.where(bad, -jnp.inf, s)
    m = jnp.max(s, axis=-1, keepdims=True)
    p = jnp.exp(s - m)
    denom = jnp.sum(p, axis=-1, keepdims=True)
    o = lax.dot_general(p, vw, (((1,), (0,)), ((), ())),
                        preferred_element_type=jnp.float32)
    o_ref[0] = o / denom


def _attention(q3, k3, v3, half, bq=128):
    NHl, T, HDl = q3.shape
    win = half + bq
    body = functools.partial(_attn_body, half=half, bq=bq, win=win)
    return pl.pallas_call(
        body,
        grid=(NHl, T // bq),
        in_specs=[
            pl.BlockSpec((1, bq, HDl), lambda h, i: (h, i, 0)),
            pl.BlockSpec((1, T, HDl), lambda h, i: (h, 0, 0)),
            pl.BlockSpec((1, T, HDl), lambda h, i: (h, 0, 0)),
        ],
        out_specs=pl.BlockSpec((1, bq, HDl), lambda h, i: (h, i, 0)),
        out_shape=jax.ShapeDtypeStruct((NHl, T, HDl), jnp.float32),
        interpret=_I,
    )(q3, k3, v3)


def _postattn_body(ao_ref, wo_ref, bo_ref, sc_ref, rw_ref, rb_ref,
                   xa_ref, g_ref):
    E = rw_ref.shape[0]
    dn = (((1,), (1,)), ((), ()))
    xa = (lax.dot_general(ao_ref[...], wo_ref[...], dn,
                          preferred_element_type=jnp.float32) + bo_ref[...])
    xa_ref[...] = xa
    hn = _rms(xa, sc_ref[...])
    lg = (lax.dot_general(hn, rw_ref[...], dn,
                          preferred_element_type=jnp.float32) + rb_ref[0])
    col = lax.broadcasted_iota(jnp.int32, lg.shape, 1)
    m1 = jnp.max(lg, axis=-1, keepdims=True)
    i1 = jnp.min(jnp.where(lg == m1, col, E), axis=-1, keepdims=True)
    lg2 = jnp.where(col == i1, -jnp.inf, lg)
    m2 = jnp.max(lg2, axis=-1, keepdims=True)
    i2 = jnp.min(jnp.where(lg2 == m2, col, E), axis=-1, keepdims=True)
    keep = (col == i1) | (col == i2)
    sp = jnp.where(keep, lg, -jnp.inf)
    p = jnp.exp(sp - m1)
    g_ref[...] = p / jnp.sum(p, axis=-1, keepdims=True)


def _postattn(ao, wo, bo, scale, rw, rb2, blk=256):
    T, D = ao.shape
    E = rw.shape[0]
    return pl.pallas_call(
        _postattn_body,
        grid=(T // blk,),
        in_specs=[
            pl.BlockSpec((blk, D), lambda i: (i, 0)),
            pl.BlockSpec((D, D), lambda i: (0, 0)),
            pl.BlockSpec((D,), lambda i: (0,)),
            pl.BlockSpec((D,), lambda i: (0,)),
            pl.BlockSpec((E, D), lambda i: (0, 0)),
            pl.BlockSpec((1, E), lambda i: (0, 0)),
        ],
        out_specs=[
            pl.BlockSpec((blk, D), lambda i: (i, 0)),
            pl.BlockSpec((blk, E), lambda i: (i, 0)),
        ],
        out_shape=[
            jax.ShapeDtypeStruct((T, D), jnp.float32),
            jax.ShapeDtypeStruct((T, E), jnp.float32),
        ],
        interpret=_I,
    )(ao, wo, bo, scale, rw, rb2)


def _moe_body(xa_ref, g_ref, sc_ref, w1_ref, b1_ref, w2_ref, b2_ref, o_ref):
    e = pl.program_id(1)
    f = pl.program_id(2)
    xa = xa_ref[...]
    hn = _rms(xa, sc_ref[...])
    h = lax.dot_general(hn, w1_ref[0], (((1,), (0,)), ((), ())),
                        preferred_element_type=jnp.float32) + b1_ref[0, 0]
    h = h * jax.nn.sigmoid(h)
    part = lax.dot_general(h, w2_ref[0], (((1,), (0,)), ((), ())),
                           preferred_element_type=jnp.float32)
    g = g_ref[...]
    col = lax.broadcasted_iota(jnp.int32, g.shape, 1)
    ge = jnp.sum(jnp.where(col == e, g, 0.0), axis=-1, keepdims=True)
    contrib = part * ge
    contrib = contrib + jnp.where(f == 0, 1.0, 0.0) * (ge * b2_ref[0, 0])
    first = (e == 0) & (f == 0)

    @pl.when(first)
    def _():
        o_ref[...] = xa + contrib

    @pl.when(jnp.logical_not(first))
    def _():
        o_ref[...] = o_ref[...] + contrib


def _moe(xa, g, scale, W1, b1, W2, b2, blk=256, bf=1024):
    T, D = xa.shape
    E, _, FF = W1.shape
    return pl.pallas_call(
        _moe_body,
        grid=(T // blk, E, FF // bf),
        in_specs=[
            pl.BlockSpec((blk, D), lambda i, e, f: (i, 0)),
            pl.BlockSpec((blk, E), lambda i, e, f: (i, 0)),
            pl.BlockSpec((D,), lambda i, e, f: (0,)),
            pl.BlockSpec((1, D, bf), lambda i, e, f: (e, 0, f)),
            pl.BlockSpec((1, 1, bf), lambda i, e, f: (e, 0, f)),
            pl.BlockSpec((1, bf, D), lambda i, e, f: (e, f, 0)),
            pl.BlockSpec((1, 1, D), lambda i, e, f: (e, 0, 0)),
        ],
        out_specs=pl.BlockSpec((blk, D), lambda i, e, f: (i, 0)),
        out_shape=jax.ShapeDtypeStruct((T, D), jnp.float32),
        interpret=_I,
    )(xa, g, scale, W1, b1.reshape(E, 1, FF), W2, b2.reshape(E, 1, D))


def _lm_body(x_ref, sc_ref, w_ref, b_ref, o_ref):
    xn = _rms(x_ref[...], sc_ref[...])
    o_ref[...] = (lax.dot_general(xn, w_ref[...], (((1,), (1,)), ((), ())),
                                  preferred_element_type=jnp.float32) + b_ref[0])


def _lm_head(x2, scale, lm_w, lm_b2, bv=1024):
    T, D = x2.shape
    Vm = lm_w.shape[0]
    return pl.pallas_call(
        _lm_body,
        grid=(pl.cdiv(Vm, bv),),
        in_specs=[
            pl.BlockSpec((T, D), lambda i: (0, 0)),
            pl.BlockSpec((D,), lambda i: (0,)),
            pl.BlockSpec((bv, D), lambda i: (i, 0)),
            pl.BlockSpec((1, bv), lambda i: (0, i)),
        ],
        out_specs=pl.BlockSpec((T, bv), lambda i: (0, i)),
        out_shape=jax.ShapeDtypeStruct((T, Vm), jnp.float32),
        interpret=_I,
    )(x2, scale, lm_w, lm_b2)


def kernel(idx, emb_table, rms1_scale, in_proj_w, in_proj_b, out_proj_w,
           out_proj_b, router_w, router_b, W1, b1, W2, b2, rms_final_scale,
           lm_w, lm_b):
    B, T = idx.shape
    V, D = emb_table.shape
    half = T // 2

    theta = 1.0 / (10000.0 ** (jnp.arange(0, HD, 2, dtype=jnp.float32) / HD))
    ang = jnp.arange(T, dtype=jnp.float32)[:, None] * theta[None, :]
    cosv = jnp.cos(ang)
    sinv = jnp.sin(ang)
    C = jnp.tile(jnp.repeat(cosv, 2, axis=1), (1, NH))
    S2 = jnp.tile(jnp.stack([sinv, -sinv], axis=-1).reshape(T, HD), (1, NH))

    Wq, Wk, Wv = in_proj_w[:D], in_proj_w[D:2 * D], in_proj_w[2 * D:]
    bq, bk, bv_ = in_proj_b[:D], in_proj_b[D:2 * D], in_proj_b[2 * D:]
    swap = jnp.arange(D).reshape(D // 2, 2)[:, ::-1].reshape(D)
    wqk = jnp.concatenate([Wq, Wk], axis=0)
    wqks = wqk[:, swap]
    bqk = jnp.concatenate([bq, bk], axis=0)

    x = _emb_gather(emb_table, idx.reshape(T))
    q, k, v = _qkv(x, rms1_scale, C, S2, wqk, wqks, Wv, bqk, bv_)
    q3 = q.reshape(T, NH, HD).transpose(1, 0, 2)
    k3 = k.reshape(T, NH, HD).transpose(1, 0, 2)
    v3 = v.reshape(T, NH, HD).transpose(1, 0, 2)
    ao3 = _attention(q3, k3, v3, half)
    ao = ao3.transpose(1, 0, 2).reshape(T, D)
    xa, g = _postattn(ao, out_proj_w, out_proj_b, rms1_scale, router_w,
                      router_b.reshape(1, -1))
    x2 = _moe(xa, g, rms1_scale, W1, b1, W2, b2)
    logits = _lm_head(x2, rms_final_scale, lm_w, lm_b.reshape(1, -1))
    return logits.reshape(B, T, V - 1)
